# baseline (speedup 1.0000x reference)
import functools
import math

import jax
import jax.numpy as jnp
from jax import lax
from jax.experimental import pallas as pl
from jax.experimental.pallas import tpu as pltpu
from jax.experimental.pallas import tpu_sc as plsc

_PAD_IDX = 0
_CONFIDENCE = 0.9
_CLOGC = _CONFIDENCE * math.log(_CONFIDENCE)

_LANES = 16
_BLK = 4096


def _sc_gather_oh(oh_flat, tgt):
    n = tgt.shape[0]
    mesh = plsc.VectorSubcoreMesh(core_axis_name="c", subcore_axis_name="s")

    @functools.partial(
        pl.kernel,
        mesh=mesh,
        out_type=jax.ShapeDtypeStruct((n,), jnp.float32),
        scratch_types=[
            pltpu.VMEM((_LANES,), jnp.int32),
            pltpu.VMEM((_LANES,), jnp.float32),
            pltpu.SemaphoreType.DMA,
        ],
    )
    def k(oh_hbm, tgt_hbm, out_hbm, tgt_v, val_v, sem):
        wid = lax.axis_index("s") * 2 + lax.axis_index("c")
        base = wid * _LANES
        pltpu.sync_copy(tgt_hbm.at[pl.ds(base, _LANES)], tgt_v)
        pltpu.async_copy(oh_hbm.at[tgt_v], val_v, sem).wait()
        pltpu.sync_copy(val_v, out_hbm.at[pl.ds(base, _LANES)])

    return k(oh_flat, tgt)


def _tc_partial(output, one_hot, tgt):
    n, v = output.shape
    nb = pl.cdiv(v, _BLK)

    def body(x_ref, w_ref, tgt_ref, rowp_ref, tvt_ref, acc_ref, tacc_ref,
             h_ref):
        i = pl.program_id(0)

        @pl.when(i == 0)
        def _init():
            acc_ref[...] = jnp.zeros_like(acc_ref)
            tacc_ref[...] = jnp.zeros_like(tacc_ref)
            h_ref[0] = 0.0

        col = lax.broadcasted_iota(jnp.int32, (1, _BLK), 1) + i * _BLK
        validc = col < v
        w = jnp.where(validc, w_ref[...], 0.0)
        x = x_ref[...]

        acc_ref[...] += jax.lax.dot_general(
            x, w, (((1,), (1,)), ((), ())),
            preferred_element_type=jnp.float32)

        tmask = col == tgt_ref[...]
        tacc_ref[...] += jnp.sum(jnp.where(tmask, x, 0.0), axis=1,
                                 keepdims=True)

        h_ref[0] += jnp.sum(
            jnp.where(w > 0, w * jnp.log(jnp.where(w > 0, w, 1.0)), 0.0))

        @pl.when(i == nb - 1)
        def _fin():
            tvt_ref[...] = tacc_ref[...]
            valid = tgt_ref[...] != _PAD_IDX
            rowp_ref[...] = jnp.where(
                valid, h_ref[0] - acc_ref[...] + _CLOGC, 0.0)

    return pl.pallas_call(
        body,
        grid=(nb,),
        in_specs=[
            pl.BlockSpec((n, _BLK), lambda i: (0, i)),
            pl.BlockSpec((1, _BLK), lambda i: (0, i)),
            pl.BlockSpec((n, 1), lambda i: (0, 0)),
        ],
        out_specs=[
            pl.BlockSpec((n, 1), lambda i: (0, 0)),
            pl.BlockSpec((n, 1), lambda i: (0, 0)),
        ],
        out_shape=[
            jax.ShapeDtypeStruct((n, 1), jnp.float32),
            jax.ShapeDtypeStruct((n, 1), jnp.float32),
        ],
        scratch_shapes=[
            pltpu.VMEM((n, 1), jnp.float32),
            pltpu.VMEM((n, 1), jnp.float32),
            pltpu.SMEM((1,), jnp.float32),
        ],
    )(output, one_hot, tgt)


def kernel(output, target, one_hot):
    n, v = output.shape
    nb = (v + _BLK - 1) // _BLK
    last = (nb - 1) * _BLK
    tgt = target.astype(jnp.int32)
    oht = _sc_gather_oh(one_hot.reshape(v), tgt)
    rowp, tvt = _tc_partial(output, one_hot, tgt.reshape(n, 1))
    tv = tvt.reshape(n)
    valid = tgt != _PAD_IDX
    g_oh = jnp.where(oht > 0, oht * jnp.log(jnp.where(oht > 0, oht, 1.0)), 0.0)
    corr = jnp.where(valid, (oht - _CONFIDENCE) * tv - g_oh, 0.0)
    return jnp.sum(rowp) + jnp.sum(corr)
